# baseline (speedup 1.0000x reference)
import functools

import jax
import jax.numpy as jnp
from jax import lax
from jax.experimental import pallas as pl
from jax.experimental.pallas import tpu as pltpu
from jax.experimental.pallas import tpu_sc as plsc

N = 10000
E = 320000
D = 128

NC = 2
NS = 16
NW = NC * NS
EPW = E // NW
CH = 112
NCH = -(-EPW // CH)
EPWP = NCH * CH
NPAD = 10240
RPS = NPAD // NS
ZCH = 128


def _sc_agg_body(h_hbm, src_hbm, dst_hbm, out_hbm,
                 sv0, sv1, sv2, sv3, dv0, dv1, dv2, dv3,
                 rows0, rows1, rows2,
                 si0, si1, si2, si3, di0, di1, di2, di3,
                 sg0, sg1, sg2, ss0, ss1, ss2, agg_sh):
    sv = [sv0, sv1, sv2, sv3]
    dv = [dv0, dv1, dv2, dv3]
    si = [si0, si1, si2, si3]
    di = [di0, di1, di2, di3]
    rows = [rows0, rows1, rows2]
    sg = [sg0, sg1, sg2]
    ss = [ss0, ss1, ss2]

    cid = lax.axis_index("c")
    sid = lax.axis_index("s")
    wid = sid * NC + cid
    base = wid * EPWP

    zero16 = jnp.zeros((16,), jnp.float32)

    @pl.loop(0, 80)
    def _(r):
        @pl.loop(0, D, step=16)
        def _(c):
            rows0[r, pl.ds(c, 16)] = zero16

    for k in range(8):
        r0 = sid * RPS + k * 80
        pltpu.sync_copy(rows0.at[pl.ds(0, 80)], agg_sh.at[pl.ds(r0, 80)])

    plsc.subcore_barrier()

    def start_idx(c, s4):
        pltpu.async_copy(src_hbm.at[pl.ds(base + c * CH, CH)], sv[s4], si[s4])
        pltpu.async_copy(dst_hbm.at[pl.ds(base + c * CH, CH)], dv[s4], di[s4])

    def wait_idx(c, s4):
        pltpu.make_async_copy(src_hbm.at[pl.ds(base + c * CH, CH)], sv[s4], si[s4]).wait()
        pltpu.make_async_copy(dst_hbm.at[pl.ds(base + c * CH, CH)], dv[s4], di[s4]).wait()

    def start_gather(s4, s3):
        pltpu.async_copy(h_hbm.at[sv[s4]], rows[s3], sg[s3])

    def wait_gather(s4, s3):
        pltpu.make_async_copy(h_hbm.at[sv[s4]], rows[s3], sg[s3]).wait()

    def start_scatter(s4, s3):
        pltpu.async_copy(rows[s3], agg_sh.at[dv[s4]], ss[s3], add=True)

    def wait_scatter(s4, s3):
        pltpu.make_async_copy(rows[s3], agg_sh.at[dv[s4]], ss[s3]).wait()

    def chunk_step(c, cm, drain=True, pf_idx=True):
        if drain:
            wait_scatter((cm - 2) % 4, (cm - 2) % 2)
        if pf_idx:
            start_idx(c + 2, (cm + 2) % 4)
        wait_idx(c, cm % 4)
        start_gather(cm % 4, cm % 2)
        wait_gather(cm % 4, cm % 2)
        start_scatter(cm % 4, cm % 2)

    start_idx(0, 0)
    start_idx(1, 1)
    chunk_step(0, 0, drain=False)
    chunk_step(1, 1, drain=False)

    @pl.loop(2, 86, step=12)
    def _(c0):
        for b in range(12):
            chunk_step(c0 + b, 2 + b)

    chunk_step(86, 86)
    chunk_step(87, 87)
    chunk_step(88, 88, pf_idx=False)
    chunk_step(89, 89, pf_idx=False)
    wait_scatter(88 % 4, 88 % 2)
    wait_scatter(89 % 4, 89 % 2)

    plsc.subcore_barrier()

    for k in range(5):
        r0 = sid * RPS + k * ZCH
        pltpu.sync_copy(agg_sh.at[pl.ds(r0, ZCH)], out_hbm.at[cid, pl.ds(r0, ZCH)])


@functools.cache
def _get_sc_aggregate():
    mesh = plsc.VectorSubcoreMesh(core_axis_name="c", subcore_axis_name="s",
                                  num_cores=NC, num_subcores=NS)
    return pl.kernel(
        _sc_agg_body,
        out_type=jax.ShapeDtypeStruct((NC, NPAD, D), jnp.float32),
        mesh=mesh,
        scratch_types=(
            [pltpu.VMEM((CH,), jnp.int32) for _ in range(8)]
            + [pltpu.VMEM((CH, D), jnp.float32) for _ in range(3)]
            + [pltpu.SemaphoreType.DMA for _ in range(14)]
            + [pltpu.VMEM_SHARED((NPAD, D), jnp.float32)]
        ),
    )


def _sc_aggregate(h, src2d, dst2d):
    return _get_sc_aggregate()(h, src2d, dst2d)


def _mlp_block(h_ref, agg_ref, w1_ref, b1_ref, w2_ref, b2_ref, o_ref, *, relu_out):
    z = h_ref[...] + agg_ref[0] + agg_ref[1]
    t = jnp.dot(z, w1_ref[...], preferred_element_type=jnp.float32) + b1_ref[...]
    t = jnp.maximum(t, 0.0)
    o = jnp.dot(t, w2_ref[...], preferred_element_type=jnp.float32) + b2_ref[...]
    if relu_out:
        o = jnp.maximum(o, 0.0)
    o_ref[...] = o


def _mlp_final_block(h_ref, agg_ref, w1_ref, b1_ref, w2_ref, b2_ref,
                     wo_ref, bo_ref, o_ref):
    z = h_ref[...] + agg_ref[0] + agg_ref[1]
    t = jnp.dot(z, w1_ref[...], preferred_element_type=jnp.float32) + b1_ref[...]
    t = jnp.maximum(t, 0.0)
    o = jnp.dot(t, w2_ref[...], preferred_element_type=jnp.float32) + b2_ref[...]
    o_ref[...] = jnp.dot(o, wo_ref[...], preferred_element_type=jnp.float32) + bo_ref[...]


_BM = 1000

_row_spec = pl.BlockSpec((_BM, D), lambda i: (i, 0))
_agg_spec = pl.BlockSpec((NC, _BM, D), lambda i: (0, i, 0))
_w_spec = pl.BlockSpec((D, D), lambda i: (0, 0))
_b_spec = pl.BlockSpec((1, D), lambda i: (0, 0))


def _tc_mlp(h, agg, w1, b1, w2, b2, relu_out):
    return pl.pallas_call(
        functools.partial(_mlp_block, relu_out=relu_out),
        grid=(N // _BM,),
        in_specs=[_row_spec, _agg_spec, _w_spec, _b_spec, _w_spec, _b_spec],
        out_specs=_row_spec,
        out_shape=jax.ShapeDtypeStruct((N, D), jnp.float32),
    )(h, agg, w1, b1.reshape(1, D), w2, b2.reshape(1, D))


def _tc_mlp_final(h, agg, w1, b1, w2, b2, wo, bo):
    return pl.pallas_call(
        _mlp_final_block,
        grid=(N // _BM,),
        in_specs=[_row_spec, _agg_spec, _w_spec, _b_spec, _w_spec, _b_spec,
                  _w_spec, _b_spec],
        out_specs=_row_spec,
        out_shape=jax.ShapeDtypeStruct((N, D), jnp.float32),
    )(h, agg, w1, b1.reshape(1, D), w2, b2.reshape(1, D), wo, bo.reshape(1, D))


def kernel(x, edge_index,
           m0c0W1, m0c0b1, m0c0W2, m0c0b2,
           m0c1W1, m0c1b1, m0c1W2, m0c1b2,
           m1c0W1, m1c0b1, m1c0W2, m1c0b2,
           m1c1W1, m1c1b1, m1c1W2, m1c1b2,
           Wout, bout):
    srcw = edge_index[0].reshape(NW, EPW)
    dstw = edge_index[1].reshape(NW, EPW)
    pad_s = jnp.zeros((NW, EPWP - EPW), jnp.int32)
    pad_d = jnp.broadcast_to(N + jnp.arange(EPWP - EPW, dtype=jnp.int32),
                             (NW, EPWP - EPW))
    src1 = jnp.concatenate([srcw, pad_s], axis=1).reshape(-1)
    dst1 = jnp.concatenate([dstw, pad_d], axis=1).reshape(-1)
    params = [
        (m0c0W1, m0c0b1, m0c0W2, m0c0b2),
        (m0c1W1, m0c1b1, m0c1W2, m0c1b2),
        (m1c0W1, m1c0b1, m1c0W2, m1c0b2),
        (m1c1W1, m1c1b1, m1c1W2, m1c1b2),
    ]
    h = x
    for l in range(3):
        w1, b1, w2, b2 = params[l]
        agg = _sc_aggregate(h, src1, dst1)
        h = _tc_mlp(h, agg, w1, b1, w2, b2, relu_out=(l in (0, 2)))
    w1, b1, w2, b2 = params[3]
    agg = _sc_aggregate(h, src1, dst1)
    return _tc_mlp_final(h, agg, w1, b1, w2, b2, Wout, bout)


# speedup vs baseline: 1.5893x; 1.5893x over previous
import functools

import jax
import jax.numpy as jnp
from jax import lax
from jax.experimental import pallas as pl
from jax.experimental.pallas import tpu as pltpu
from jax.experimental.pallas import tpu_sc as plsc

N = 10000
E = 320000
D = 128

NC = 2
NS = 16
NW = NC * NS
EPW = E // NW
CH = 112
NCH = -(-EPW // CH)
EPWP = NCH * CH
NPAD = 10240
RPS = NPAD // NS
ZCH = 128


def _sc_agg_body(h_hbm, src_hbm, dst_hbm, out_hbm,
                 sv0, sv1, sv2, sv3, dv0, dv1, dv2, dv3,
                 rows0, rows1, rows2,
                 si0, si1, si2, si3, di0, di1, di2, di3,
                 sg0, sg1, sg2, ss0, ss1, ss2, agg_sh):
    sv = [sv0, sv1, sv2, sv3]
    dv = [dv0, dv1, dv2, dv3]
    si = [si0, si1, si2, si3]
    di = [di0, di1, di2, di3]
    rows = [rows0, rows1, rows2]
    sg = [sg0, sg1, sg2]
    ss = [ss0, ss1, ss2]

    cid = lax.axis_index("c")
    sid = lax.axis_index("s")
    wid = sid * NC + cid
    base = wid * EPWP

    zero16 = jnp.zeros((16,), jnp.float32)

    @pl.loop(0, 80)
    def _(r):
        @pl.loop(0, D, step=16)
        def _(c):
            rows0[r, pl.ds(c, 16)] = zero16

    for k in range(8):
        r0 = sid * RPS + k * 80
        pltpu.sync_copy(rows0.at[pl.ds(0, 80)], agg_sh.at[pl.ds(r0, 80)])

    plsc.subcore_barrier()

    def start_idx(c, s4):
        pltpu.async_copy(src_hbm.at[pl.ds(base + c * CH, CH)], sv[s4], si[s4])
        pltpu.async_copy(dst_hbm.at[pl.ds(base + c * CH, CH)], dv[s4], di[s4])

    def wait_idx(c, s4):
        pltpu.make_async_copy(src_hbm.at[pl.ds(base + c * CH, CH)], sv[s4], si[s4]).wait()
        pltpu.make_async_copy(dst_hbm.at[pl.ds(base + c * CH, CH)], dv[s4], di[s4]).wait()

    def start_gather(s4, s3):
        pltpu.async_copy(h_hbm.at[sv[s4]], rows[s3], sg[s3])

    def wait_gather(s4, s3):
        pltpu.make_async_copy(h_hbm.at[sv[s4]], rows[s3], sg[s3]).wait()

    def start_scatter(s4, s3):
        pltpu.async_copy(rows[s3], agg_sh.at[dv[s4]], ss[s3], add=True)

    def wait_scatter(s4, s3):
        pltpu.make_async_copy(rows[s3], agg_sh.at[dv[s4]], ss[s3]).wait()

    def chunk_step(c, cm, drain=True, pf_idx=True):
        if drain:
            wait_scatter((cm - 2) % 4, (cm - 2) % 2)
        if pf_idx:
            start_idx(c + 2, (cm + 2) % 4)
        wait_idx(c, cm % 4)
        start_gather(cm % 4, cm % 2)
        wait_gather(cm % 4, cm % 2)
        start_scatter(cm % 4, cm % 2)

    start_idx(0, 0)
    start_idx(1, 1)
    chunk_step(0, 0, drain=False)
    chunk_step(1, 1, drain=False)

    @pl.loop(2, 86, step=12)
    def _(c0):
        for b in range(12):
            chunk_step(c0 + b, 2 + b)

    chunk_step(86, 86)
    chunk_step(87, 87)
    chunk_step(88, 88, pf_idx=False)
    chunk_step(89, 89, pf_idx=False)
    wait_scatter(88 % 4, 88 % 2)
    wait_scatter(89 % 4, 89 % 2)

    plsc.subcore_barrier()

    for k in range(5):
        r0 = sid * RPS + k * ZCH
        pltpu.sync_copy(agg_sh.at[pl.ds(r0, ZCH)], out_hbm.at[cid, pl.ds(r0, ZCH)])


@functools.cache
def _get_sc_aggregate():
    mesh = plsc.VectorSubcoreMesh(core_axis_name="c", subcore_axis_name="s",
                                  num_cores=NC, num_subcores=NS)
    return pl.kernel(
        _sc_agg_body,
        out_type=jax.ShapeDtypeStruct((NC, NPAD, D), jnp.float32),
        mesh=mesh,
        scratch_types=(
            [pltpu.VMEM((CH,), jnp.int32) for _ in range(8)]
            + [pltpu.VMEM((CH, D), jnp.float32) for _ in range(3)]
            + [pltpu.SemaphoreType.DMA for _ in range(14)]
            + [pltpu.VMEM_SHARED((NPAD, D), jnp.float32)]
        ),
    )


def _sc_aggregate(h, src2d, dst2d):
    return _get_sc_aggregate()(h, src2d, dst2d)


def _mlp_block(h_ref, agg_ref, w1_ref, b1_ref, w2_ref, b2_ref, o_ref, *, relu_out):
    z = h_ref[...] + agg_ref[0] + agg_ref[1]
    t = jnp.dot(z, w1_ref[...], preferred_element_type=jnp.float32) + b1_ref[...]
    t = jnp.maximum(t, 0.0)
    o = jnp.dot(t, w2_ref[...], preferred_element_type=jnp.float32) + b2_ref[...]
    if relu_out:
        o = jnp.maximum(o, 0.0)
    o_ref[...] = o


def _mlp_final_block(h_ref, agg_ref, w1_ref, b1_ref, w2_ref, b2_ref,
                     wo_ref, bo_ref, o_ref):
    z = h_ref[...] + agg_ref[0] + agg_ref[1]
    t = jnp.dot(z, w1_ref[...], preferred_element_type=jnp.float32) + b1_ref[...]
    t = jnp.maximum(t, 0.0)
    o = jnp.dot(t, w2_ref[...], preferred_element_type=jnp.float32) + b2_ref[...]
    o_ref[...] = jnp.dot(o, wo_ref[...], preferred_element_type=jnp.float32) + bo_ref[...]


_BM = 1000

_row_spec = pl.BlockSpec((_BM, D), lambda i: (i, 0))
_agg_spec = pl.BlockSpec((NC, _BM, D), lambda i: (0, i, 0))
_w_spec = pl.BlockSpec((D, D), lambda i: (0, 0))
_b_spec = pl.BlockSpec((1, D), lambda i: (0, 0))


def _tc_mlp(h, agg, w1, b1, w2, b2, relu_out):
    return pl.pallas_call(
        functools.partial(_mlp_block, relu_out=relu_out),
        grid=(N // _BM,),
        in_specs=[_row_spec, _agg_spec, _w_spec, _b_spec, _w_spec, _b_spec],
        out_specs=_row_spec,
        out_shape=jax.ShapeDtypeStruct((N, D), jnp.float32),
    )(h, agg, w1, b1.reshape(1, D), w2, b2.reshape(1, D))


def _tc_mlp_final(h, agg, w1, b1, w2, b2, wo, bo):
    return pl.pallas_call(
        _mlp_final_block,
        grid=(N // _BM,),
        in_specs=[_row_spec, _agg_spec, _w_spec, _b_spec, _w_spec, _b_spec,
                  _w_spec, _b_spec],
        out_specs=_row_spec,
        out_shape=jax.ShapeDtypeStruct((N, D), jnp.float32),
    )(h, agg, w1, b1.reshape(1, D), w2, b2.reshape(1, D), wo, bo.reshape(1, D))


def kernel(x, edge_index,
           m0c0W1, m0c0b1, m0c0W2, m0c0b2,
           m0c1W1, m0c1b1, m0c1W2, m0c1b2,
           m1c0W1, m1c0b1, m1c0W2, m1c0b2,
           m1c1W1, m1c1b1, m1c1W2, m1c1b2,
           Wout, bout):
    srcw = edge_index[0].reshape(NW, EPW)
    dstw = edge_index[1].reshape(NW, EPW)
    pad_s = jnp.broadcast_to(jnp.arange(EPWP - EPW, dtype=jnp.int32),
                             (NW, EPWP - EPW))
    pad_d = jnp.broadcast_to(N + jnp.arange(EPWP - EPW, dtype=jnp.int32),
                             (NW, EPWP - EPW))
    src1 = jnp.concatenate([srcw, pad_s], axis=1).reshape(-1)
    dst1 = jnp.concatenate([dstw, pad_d], axis=1).reshape(-1)
    params = [
        (m0c0W1, m0c0b1, m0c0W2, m0c0b2),
        (m0c1W1, m0c1b1, m0c1W2, m0c1b2),
        (m1c0W1, m1c0b1, m1c0W2, m1c0b2),
        (m1c1W1, m1c1b1, m1c1W2, m1c1b2),
    ]
    h = x
    for l in range(3):
        w1, b1, w2, b2 = params[l]
        agg = _sc_aggregate(h, src1, dst1)
        h = _tc_mlp(h, agg, w1, b1, w2, b2, relu_out=(l in (0, 2)))
    w1, b1, w2, b2 = params[3]
    agg = _sc_aggregate(h, src1, dst1)
    return _tc_mlp_final(h, agg, w1, b1, w2, b2, Wout, bout)


# speedup vs baseline: 2.1018x; 1.3224x over previous
import functools

import jax
import jax.numpy as jnp
from jax import lax
from jax.experimental import pallas as pl
from jax.experimental.pallas import tpu as pltpu
from jax.experimental.pallas import tpu_sc as plsc

N = 10000
E = 320000
D = 128

NC = 2
NS = 16
NW = NC * NS
EPW = E // NW
CH = 112
NCH = -(-EPW // CH)
EPWP = NCH * CH
NPAD = 10240
RPS = NPAD // NS
ZCH = 128


def _sc_agg_body(h_hbm, src_hbm, dst_hbm, out_hbm,
                 sv0, sv1, sv2, sv3, dv0, dv1, dv2, dv3,
                 rows0, rows1, rows2,
                 si0, si1, si2, si3, di0, di1, di2, di3,
                 sg0, sg1, sg2, ss0, ss1, ss2, agg_sh):
    sv = [sv0, sv1, sv2, sv3]
    dv = [dv0, dv1, dv2, dv3]
    si = [si0, si1, si2, si3]
    di = [di0, di1, di2, di3]
    rows = [rows0, rows1, rows2]
    sg = [sg0, sg1, sg2]
    ss = [ss0, ss1, ss2]

    cid = lax.axis_index("c")
    sid = lax.axis_index("s")
    wid = sid * NC + cid
    base = wid * EPWP

    zero16 = jnp.zeros((16,), jnp.float32)

    @pl.loop(0, 80)
    def _(r):
        @pl.loop(0, D, step=16)
        def _(c):
            rows0[r, pl.ds(c, 16)] = zero16

    for k in range(8):
        r0 = sid * RPS + k * 80
        pltpu.sync_copy(rows0.at[pl.ds(0, 80)], agg_sh.at[pl.ds(r0, 80)])

    plsc.subcore_barrier()

    def start_idx(c, s4):
        pltpu.async_copy(src_hbm.at[pl.ds(base + c * CH, CH)], sv[s4], si[s4])
        pltpu.async_copy(dst_hbm.at[pl.ds(base + c * CH, CH)], dv[s4], di[s4])

    def wait_idx(c, s4):
        pltpu.make_async_copy(src_hbm.at[pl.ds(base + c * CH, CH)], sv[s4], si[s4]).wait()
        pltpu.make_async_copy(dst_hbm.at[pl.ds(base + c * CH, CH)], dv[s4], di[s4]).wait()

    def start_gather(s4, s3):
        pltpu.async_copy(h_hbm.at[sv[s4]], rows[s3], sg[s3])

    def wait_gather(s4, s3):
        pltpu.make_async_copy(h_hbm.at[sv[s4]], rows[s3], sg[s3]).wait()

    def start_scatter(s4, s3):
        pltpu.async_copy(rows[s3], agg_sh.at[dv[s4]], ss[s3], add=True)

    def wait_scatter(s4, s3):
        pltpu.make_async_copy(rows[s3], agg_sh.at[dv[s4]], ss[s3]).wait()

    def chunk_step(c, cm, drain=True, pf_idx=True, next_gather=True):
        if drain:
            wait_scatter((cm - 2) % 4, (cm - 2) % 3)
        if pf_idx:
            start_idx(c + 2, (cm + 2) % 4)
        if next_gather:
            wait_idx(c + 1, (cm + 1) % 4)
            start_gather((cm + 1) % 4, (cm + 1) % 3)
        wait_gather(cm % 4, cm % 3)
        start_scatter(cm % 4, cm % 3)

    start_idx(0, 0)
    start_idx(1, 1)
    wait_idx(0, 0)
    start_gather(0, 0)
    chunk_step(0, 0, drain=False)
    chunk_step(1, 1, drain=False)

    @pl.loop(2, 86, step=12)
    def _(c0):
        for b in range(12):
            chunk_step(c0 + b, 2 + b)

    chunk_step(86, 86)
    chunk_step(87, 87)
    chunk_step(88, 88, pf_idx=False)
    chunk_step(89, 89, pf_idx=False, next_gather=False)
    wait_scatter(88 % 4, 88 % 3)
    wait_scatter(89 % 4, 89 % 3)

    plsc.subcore_barrier()

    for k in range(5):
        r0 = sid * RPS + k * ZCH
        pltpu.sync_copy(agg_sh.at[pl.ds(r0, ZCH)], out_hbm.at[cid, pl.ds(r0, ZCH)])


@functools.cache
def _get_sc_aggregate():
    mesh = plsc.VectorSubcoreMesh(core_axis_name="c", subcore_axis_name="s",
                                  num_cores=NC, num_subcores=NS)
    return pl.kernel(
        _sc_agg_body,
        out_type=jax.ShapeDtypeStruct((NC, NPAD, D), jnp.float32),
        mesh=mesh,
        scratch_types=(
            [pltpu.VMEM((CH,), jnp.int32) for _ in range(8)]
            + [pltpu.VMEM((CH, D), jnp.float32) for _ in range(3)]
            + [pltpu.SemaphoreType.DMA for _ in range(14)]
            + [pltpu.VMEM_SHARED((NPAD, D), jnp.float32)]
        ),
    )


def _sc_aggregate(h, src2d, dst2d):
    return _get_sc_aggregate()(h, src2d, dst2d)


def _mlp_block(h_ref, agg_ref, w1_ref, b1_ref, w2_ref, b2_ref, o_ref, *, relu_out):
    z = h_ref[...] + agg_ref[0] + agg_ref[1]
    t = jnp.dot(z, w1_ref[...], preferred_element_type=jnp.float32) + b1_ref[...]
    t = jnp.maximum(t, 0.0)
    o = jnp.dot(t, w2_ref[...], preferred_element_type=jnp.float32) + b2_ref[...]
    if relu_out:
        o = jnp.maximum(o, 0.0)
    o_ref[...] = o


def _mlp_final_block(h_ref, agg_ref, w1_ref, b1_ref, w2_ref, b2_ref,
                     wo_ref, bo_ref, o_ref):
    z = h_ref[...] + agg_ref[0] + agg_ref[1]
    t = jnp.dot(z, w1_ref[...], preferred_element_type=jnp.float32) + b1_ref[...]
    t = jnp.maximum(t, 0.0)
    o = jnp.dot(t, w2_ref[...], preferred_element_type=jnp.float32) + b2_ref[...]
    o_ref[...] = jnp.dot(o, wo_ref[...], preferred_element_type=jnp.float32) + bo_ref[...]


_BM = 1000

_row_spec = pl.BlockSpec((_BM, D), lambda i: (i, 0))
_agg_spec = pl.BlockSpec((NC, _BM, D), lambda i: (0, i, 0))
_w_spec = pl.BlockSpec((D, D), lambda i: (0, 0))
_b_spec = pl.BlockSpec((1, D), lambda i: (0, 0))


def _tc_mlp(h, agg, w1, b1, w2, b2, relu_out):
    return pl.pallas_call(
        functools.partial(_mlp_block, relu_out=relu_out),
        grid=(N // _BM,),
        in_specs=[_row_spec, _agg_spec, _w_spec, _b_spec, _w_spec, _b_spec],
        out_specs=_row_spec,
        out_shape=jax.ShapeDtypeStruct((N, D), jnp.float32),
    )(h, agg, w1, b1.reshape(1, D), w2, b2.reshape(1, D))


def _tc_mlp_final(h, agg, w1, b1, w2, b2, wo, bo):
    return pl.pallas_call(
        _mlp_final_block,
        grid=(N // _BM,),
        in_specs=[_row_spec, _agg_spec, _w_spec, _b_spec, _w_spec, _b_spec,
                  _w_spec, _b_spec],
        out_specs=_row_spec,
        out_shape=jax.ShapeDtypeStruct((N, D), jnp.float32),
    )(h, agg, w1, b1.reshape(1, D), w2, b2.reshape(1, D), wo, bo.reshape(1, D))


def kernel(x, edge_index,
           m0c0W1, m0c0b1, m0c0W2, m0c0b2,
           m0c1W1, m0c1b1, m0c1W2, m0c1b2,
           m1c0W1, m1c0b1, m1c0W2, m1c0b2,
           m1c1W1, m1c1b1, m1c1W2, m1c1b2,
           Wout, bout):
    srcw = edge_index[0].reshape(NW, EPW)
    dstw = edge_index[1].reshape(NW, EPW)
    pad_s = jnp.broadcast_to(jnp.arange(EPWP - EPW, dtype=jnp.int32),
                             (NW, EPWP - EPW))
    pad_d = jnp.broadcast_to(N + jnp.arange(EPWP - EPW, dtype=jnp.int32),
                             (NW, EPWP - EPW))
    src1 = jnp.concatenate([srcw, pad_s], axis=1).reshape(-1)
    dst1 = jnp.concatenate([dstw, pad_d], axis=1).reshape(-1)
    params = [
        (m0c0W1, m0c0b1, m0c0W2, m0c0b2),
        (m0c1W1, m0c1b1, m0c1W2, m0c1b2),
        (m1c0W1, m1c0b1, m1c0W2, m1c0b2),
        (m1c1W1, m1c1b1, m1c1W2, m1c1b2),
    ]
    h = x
    for l in range(3):
        w1, b1, w2, b2 = params[l]
        agg = _sc_aggregate(h, src1, dst1)
        h = _tc_mlp(h, agg, w1, b1, w2, b2, relu_out=(l in (0, 2)))
    w1, b1, w2, b2 = params[3]
    agg = _sc_aggregate(h, src1, dst1)
    return _tc_mlp_final(h, agg, w1, b1, w2, b2, Wout, bout)


# speedup vs baseline: 2.1340x; 1.0153x over previous
import functools

import jax
import jax.numpy as jnp
from jax import lax
from jax.experimental import pallas as pl
from jax.experimental.pallas import tpu as pltpu
from jax.experimental.pallas import tpu_sc as plsc

N = 10000
E = 320000
D = 128

NC = 2
NS = 16
NW = NC * NS
EPW = E // NW
CH = 112
NCH = -(-EPW // CH)
EPWP = NCH * CH
NPAD = 10240
RPS = NPAD // NS
ZCH = 128


def _sc_agg_body(h_hbm, src_hbm, dst_hbm, out_hbm,
                 sv0, sv1, sv2, sv3, dv0, dv1, dv2, dv3,
                 rows0, rows1, rows2,
                 si0, si1, si2, si3, di0, di1, di2, di3,
                 sg0, sg1, sg2, ss0, ss1, ss2, agg_sh):
    sv = [sv0, sv1, sv2, sv3]
    dv = [dv0, dv1, dv2, dv3]
    si = [si0, si1, si2, si3]
    di = [di0, di1, di2, di3]
    rows = [rows0, rows1, rows2]
    sg = [sg0, sg1, sg2]
    ss = [ss0, ss1, ss2]

    cid = lax.axis_index("c")
    sid = lax.axis_index("s")
    wid = sid * NC + cid
    base = wid * EPWP

    zero16 = jnp.zeros((16,), jnp.float32)

    @pl.loop(0, 80)
    def _(r):
        @pl.loop(0, D, step=16)
        def _(c):
            rows2[r, pl.ds(c, 16)] = zero16

    def start_idx(c, s4):
        pltpu.async_copy(src_hbm.at[pl.ds(base + c * CH, CH)], sv[s4], si[s4])
        pltpu.async_copy(dst_hbm.at[pl.ds(base + c * CH, CH)], dv[s4], di[s4])

    def wait_idx(c, s4):
        pltpu.make_async_copy(src_hbm.at[pl.ds(base + c * CH, CH)], sv[s4], si[s4]).wait()
        pltpu.make_async_copy(dst_hbm.at[pl.ds(base + c * CH, CH)], dv[s4], di[s4]).wait()

    def start_gather(s4, s3):
        pltpu.async_copy(h_hbm.at[sv[s4]], rows[s3], sg[s3])

    def wait_gather(s4, s3):
        pltpu.make_async_copy(h_hbm.at[sv[s4]], rows[s3], sg[s3]).wait()

    def start_scatter(s4, s3):
        pltpu.async_copy(rows[s3], agg_sh.at[dv[s4]], ss[s3], add=True)

    def wait_scatter(s4, s3):
        pltpu.make_async_copy(rows[s3], agg_sh.at[dv[s4]], ss[s3]).wait()

    def chunk_step(c, cm, drain=True, pf_idx=True, next_gather=True):
        if drain:
            wait_scatter((cm - 2) % 4, (cm - 2) % 3)
        if pf_idx:
            start_idx(c + 2, (cm + 2) % 4)
        if next_gather:
            wait_idx(c + 1, (cm + 1) % 4)
            start_gather((cm + 1) % 4, (cm + 1) % 3)
        wait_gather(cm % 4, cm % 3)
        start_scatter(cm % 4, cm % 3)

    start_idx(0, 0)
    start_idx(1, 1)
    for k in range(8):
        r0 = sid * RPS + k * 80
        pltpu.async_copy(rows2.at[pl.ds(0, 80)], agg_sh.at[pl.ds(r0, 80)], ss2)
    wait_idx(0, 0)
    start_gather(0, 0)
    for k in range(8):
        r0 = sid * RPS + k * 80
        pltpu.make_async_copy(rows2.at[pl.ds(0, 80)], agg_sh.at[pl.ds(r0, 80)], ss2).wait()
    plsc.subcore_barrier()
    chunk_step(0, 0, drain=False)
    chunk_step(1, 1, drain=False)

    @pl.loop(2, 86, step=12)
    def _(c0):
        for b in range(12):
            chunk_step(c0 + b, 2 + b)

    chunk_step(86, 86)
    chunk_step(87, 87)
    chunk_step(88, 88, pf_idx=False)
    chunk_step(89, 89, pf_idx=False, next_gather=False)
    wait_scatter(88 % 4, 88 % 3)
    wait_scatter(89 % 4, 89 % 3)

    plsc.subcore_barrier()

    for k in range(5):
        r0 = sid * RPS + k * ZCH
        pltpu.async_copy(agg_sh.at[pl.ds(r0, ZCH)], out_hbm.at[cid, pl.ds(r0, ZCH)], sg0)
    for k in range(5):
        r0 = sid * RPS + k * ZCH
        pltpu.make_async_copy(agg_sh.at[pl.ds(r0, ZCH)], out_hbm.at[cid, pl.ds(r0, ZCH)], sg0).wait()


@functools.cache
def _get_sc_aggregate():
    mesh = plsc.VectorSubcoreMesh(core_axis_name="c", subcore_axis_name="s",
                                  num_cores=NC, num_subcores=NS)
    return pl.kernel(
        _sc_agg_body,
        out_type=jax.ShapeDtypeStruct((NC, NPAD, D), jnp.float32),
        mesh=mesh,
        scratch_types=(
            [pltpu.VMEM((CH,), jnp.int32) for _ in range(8)]
            + [pltpu.VMEM((CH, D), jnp.float32) for _ in range(3)]
            + [pltpu.SemaphoreType.DMA for _ in range(14)]
            + [pltpu.VMEM_SHARED((NPAD, D), jnp.float32)]
        ),
    )


def _sc_aggregate(h, src2d, dst2d):
    return _get_sc_aggregate()(h, src2d, dst2d)


def _mlp_block(h_ref, agg_ref, w1_ref, b1_ref, w2_ref, b2_ref, o_ref, *, relu_out):
    z = h_ref[...] + agg_ref[0] + agg_ref[1]
    t = jnp.dot(z, w1_ref[...], preferred_element_type=jnp.float32) + b1_ref[...]
    t = jnp.maximum(t, 0.0)
    o = jnp.dot(t, w2_ref[...], preferred_element_type=jnp.float32) + b2_ref[...]
    if relu_out:
        o = jnp.maximum(o, 0.0)
    o_ref[...] = o


def _mlp_final_block(h_ref, agg_ref, w1_ref, b1_ref, w2_ref, b2_ref,
                     wo_ref, bo_ref, o_ref):
    z = h_ref[...] + agg_ref[0] + agg_ref[1]
    t = jnp.dot(z, w1_ref[...], preferred_element_type=jnp.float32) + b1_ref[...]
    t = jnp.maximum(t, 0.0)
    o = jnp.dot(t, w2_ref[...], preferred_element_type=jnp.float32) + b2_ref[...]
    o_ref[...] = jnp.dot(o, wo_ref[...], preferred_element_type=jnp.float32) + bo_ref[...]


_BM = 1000

_row_spec = pl.BlockSpec((_BM, D), lambda i: (i, 0))
_agg_spec = pl.BlockSpec((NC, _BM, D), lambda i: (0, i, 0))
_w_spec = pl.BlockSpec((D, D), lambda i: (0, 0))
_b_spec = pl.BlockSpec((1, D), lambda i: (0, 0))


def _tc_mlp(h, agg, w1, b1, w2, b2, relu_out):
    return pl.pallas_call(
        functools.partial(_mlp_block, relu_out=relu_out),
        grid=(N // _BM,),
        in_specs=[_row_spec, _agg_spec, _w_spec, _b_spec, _w_spec, _b_spec],
        out_specs=_row_spec,
        out_shape=jax.ShapeDtypeStruct((N, D), jnp.float32),
    )(h, agg, w1, b1.reshape(1, D), w2, b2.reshape(1, D))


def _tc_mlp_final(h, agg, w1, b1, w2, b2, wo, bo):
    return pl.pallas_call(
        _mlp_final_block,
        grid=(N // _BM,),
        in_specs=[_row_spec, _agg_spec, _w_spec, _b_spec, _w_spec, _b_spec,
                  _w_spec, _b_spec],
        out_specs=_row_spec,
        out_shape=jax.ShapeDtypeStruct((N, D), jnp.float32),
    )(h, agg, w1, b1.reshape(1, D), w2, b2.reshape(1, D), wo, bo.reshape(1, D))


def kernel(x, edge_index,
           m0c0W1, m0c0b1, m0c0W2, m0c0b2,
           m0c1W1, m0c1b1, m0c1W2, m0c1b2,
           m1c0W1, m1c0b1, m1c0W2, m1c0b2,
           m1c1W1, m1c1b1, m1c1W2, m1c1b2,
           Wout, bout):
    srcw = edge_index[0].reshape(NW, EPW)
    dstw = edge_index[1].reshape(NW, EPW)
    pad_s = jnp.broadcast_to(jnp.arange(EPWP - EPW, dtype=jnp.int32),
                             (NW, EPWP - EPW))
    pad_d = jnp.broadcast_to(N + jnp.arange(EPWP - EPW, dtype=jnp.int32),
                             (NW, EPWP - EPW))
    src1 = jnp.concatenate([srcw, pad_s], axis=1).reshape(-1)
    dst1 = jnp.concatenate([dstw, pad_d], axis=1).reshape(-1)
    params = [
        (m0c0W1, m0c0b1, m0c0W2, m0c0b2),
        (m0c1W1, m0c1b1, m0c1W2, m0c1b2),
        (m1c0W1, m1c0b1, m1c0W2, m1c0b2),
        (m1c1W1, m1c1b1, m1c1W2, m1c1b2),
    ]
    h = x
    for l in range(3):
        w1, b1, w2, b2 = params[l]
        agg = _sc_aggregate(h, src1, dst1)
        h = _tc_mlp(h, agg, w1, b1, w2, b2, relu_out=(l in (0, 2)))
    w1, b1, w2, b2 = params[3]
    agg = _sc_aggregate(h, src1, dst1)
    return _tc_mlp_final(h, agg, w1, b1, w2, b2, Wout, bout)
